# Initial kernel scaffold; baseline (speedup 1.0000x reference)
import jax
import jax.numpy as jnp
from jax.experimental import pallas as pl


def kernel(x, emb, W1, b1, W2, b2):
    raise NotImplementedError("write your pallas kernel here")


# speedup vs baseline: 8.0116x; 8.0116x over previous
import functools

import jax
import jax.numpy as jnp
from jax import lax
from jax.experimental import pallas as pl
from jax.experimental.pallas import tpu as pltpu
from jax.experimental.pallas import tpu_sc as plsc

V = 100000
D = 128
C = 2
B = 4096
L = 200

NC = 2
NS = 16
NW = NC * NS
BPW = B // NW
HALF = L // 2
NLANE = 16
NCHUNK = D // NLANE

_mesh = plsc.VectorSubcoreMesh(core_axis_name="c", subcore_axis_name="s")


@functools.partial(
    pl.kernel,
    mesh=_mesh,
    out_type=jax.ShapeDtypeStruct((B, D), jnp.float32),
    scratch_types=[
        pltpu.VMEM((2 * BPW, HALF), jnp.int32),
        pltpu.VMEM((HALF, D), jnp.float32),
        pltpu.VMEM((HALF, D), jnp.float32),
        pltpu.VMEM((BPW, D), jnp.float32),
        pltpu.SemaphoreType.DMA,
        pltpu.SemaphoreType.DMA,
    ],
)
def _pool_sc(x_hbm, emb_hbm, out_hbm, idx_v, rows_a, rows_b, out_v, sem_a, sem_b):
    wid = lax.axis_index("s") * NC + lax.axis_index("c")
    base = wid * BPW
    pltpu.sync_copy(x_hbm.at[pl.ds(2 * base, 2 * BPW)], idx_v)

    def row_body(b, carry):
        ca = pltpu.async_copy(emb_hbm.at[idx_v.at[2 * b]], rows_a, sem_a)
        cb = pltpu.async_copy(emb_hbm.at[idx_v.at[2 * b + 1]], rows_b, sem_b)
        ca.wait()
        cb.wait()

        def racc(r, accs):
            return tuple(
                accs[c]
                + rows_a[r, pl.ds(c * NLANE, NLANE)]
                + rows_b[r, pl.ds(c * NLANE, NLANE)]
                for c in range(NCHUNK)
            )

        zeros = tuple(jnp.zeros((NLANE,), jnp.float32) for _ in range(NCHUNK))
        accs = lax.fori_loop(0, HALF, racc, zeros)
        for c in range(NCHUNK):
            out_v[b, pl.ds(c * NLANE, NLANE)] = accs[c]
        return carry

    lax.fori_loop(0, BPW, row_body, 0)
    pltpu.sync_copy(out_v, out_hbm.at[pl.ds(base, BPW)])


def _mlp_body(h_ref, w1_ref, b1_ref, w2_ref, b2_ref, o_ref):
    h = h_ref[...] * (1.0 / L)
    z = jnp.dot(h, w1_ref[...], preferred_element_type=jnp.float32)
    z = jnp.maximum(z + b1_ref[...], 0.0)
    o_ref[...] = jnp.dot(z, w2_ref[...], preferred_element_type=jnp.float32) + b2_ref[...]


@jax.jit
def kernel(x, emb, W1, b1, W2, b2):
    x2 = x.reshape(2 * B, HALF)
    pooled = _pool_sc(x2, emb)
    out = pl.pallas_call(
        _mlp_body,
        out_shape=jax.ShapeDtypeStruct((B, C), jnp.float32),
    )(pooled, W1, b1.reshape(1, 64), W2, b2.reshape(1, C))
    return out


# speedup vs baseline: 13.3413x; 1.6652x over previous
import functools

import jax
import jax.numpy as jnp
from jax import lax
from jax.experimental import pallas as pl
from jax.experimental.pallas import tpu as pltpu
from jax.experimental.pallas import tpu_sc as plsc

V = 100000
D = 128
C = 2
B = 4096
L = 200

NC = 2
NS = 16
NW = NC * NS
BPW = B // NW
HALF = L // 2
NLANE = 16
NCHUNK = D // NLANE

_mesh = plsc.VectorSubcoreMesh(core_axis_name="c", subcore_axis_name="s")


@functools.partial(
    pl.kernel,
    mesh=_mesh,
    out_type=jax.ShapeDtypeStruct((B, D), jnp.float32),
    scratch_types=[
        pltpu.VMEM((2 * BPW, HALF), jnp.int32),
        pltpu.VMEM((L, D), jnp.float32),
        pltpu.VMEM((L, D), jnp.float32),
        pltpu.VMEM((BPW, D), jnp.float32),
        pltpu.SemaphoreType.DMA,
        pltpu.SemaphoreType.DMA,
    ],
)
def _pool_sc(x_hbm, emb_hbm, out_hbm, idx_v, rows0, rows1, out_v, sem0, sem1):
    wid = lax.axis_index("s") * NC + lax.axis_index("c")
    base = wid * BPW
    pltpu.sync_copy(x_hbm.at[pl.ds(2 * base, 2 * BPW)], idx_v)

    rows = (rows0, rows1)
    sems = (sem0, sem1)

    def issue(slot, row):
        pltpu.async_copy(
            emb_hbm.at[idx_v.at[2 * row]], rows[slot].at[pl.ds(0, HALF)], sems[slot])
        pltpu.async_copy(
            emb_hbm.at[idx_v.at[2 * row + 1]], rows[slot].at[pl.ds(HALF, HALF)], sems[slot])

    def wait(slot):
        pltpu.make_async_copy(emb_hbm.at[pl.ds(0, L)], rows[slot], sems[slot]).wait()

    issue(0, 0)

    @pl.loop(0, BPW, step=2)
    def _row_pair(b0):
        for k in range(2):
            row = b0 + k
            nxt = row + 1

            @pl.when(nxt < BPW)
            def _prefetch():
                issue((k + 1) % 2, nxt)

            wait(k)
            buf = rows[k]

            def racc(r, accs):
                return tuple(
                    accs[c] + buf[r, pl.ds(c * NLANE, NLANE)] for c in range(NCHUNK))

            zeros = tuple(jnp.zeros((NLANE,), jnp.float32) for _ in range(NCHUNK))
            accs = lax.fori_loop(0, L, racc, zeros, unroll=4)
            for c in range(NCHUNK):
                out_v[row, pl.ds(c * NLANE, NLANE)] = accs[c]

    pltpu.sync_copy(out_v, out_hbm.at[pl.ds(base, BPW)])


def _mlp_body(h_ref, w1_ref, b1_ref, w2_ref, b2_ref, o_ref):
    h = h_ref[...] * (1.0 / L)
    z = jnp.dot(h, w1_ref[...], preferred_element_type=jnp.float32)
    z = jnp.maximum(z + b1_ref[...], 0.0)
    o_ref[...] = jnp.dot(z, w2_ref[...], preferred_element_type=jnp.float32) + b2_ref[...]


@jax.jit
def kernel(x, emb, W1, b1, W2, b2):
    x2 = x.reshape(2 * B, HALF)
    pooled = _pool_sc(x2, emb)
    out = pl.pallas_call(
        _mlp_body,
        out_shape=jax.ShapeDtypeStruct((B, C), jnp.float32),
    )(pooled, W1, b1.reshape(1, 64), W2, b2.reshape(1, C))
    return out
